# baseline (speedup 1.0000x reference)
import jax
import jax.numpy as jnp
from jax.experimental import pallas as pl
from jax.experimental.pallas import tpu as pltpu

B, S, H, G = 2, 2048, 1024, 256
IF, IC = 4096, 2048
ITOT = IF + IC + IF
BS = 1024
BI = 512
NS = (B * S) // BS
NF = IF // BI
NC = IC // BI
NI = NF + NC + NF
ROWS_PER_BATCH = S // BS


def _gate_body(x_ref, gw1_ref, gb1_ref, gw2_ref, gb2_ref, b2s_ref,
               coef_ref, b2e_ref):
    gf0 = jnp.mean(x_ref[0:S, :], axis=0, keepdims=True)
    gf1 = jnp.mean(x_ref[S:2 * S, :], axis=0, keepdims=True)
    gf = jnp.concatenate([gf0, gf1], axis=0)
    gh = jnp.maximum(
        jnp.dot(gf, gw1_ref[...], preferred_element_type=jnp.float32)
        + gb1_ref[...], 0.0)
    logits = (jnp.dot(gh, gw2_ref[...], preferred_element_type=jnp.float32)
              + gb2_ref[...])
    m = jnp.max(logits, axis=-1, keepdims=True)
    e = jnp.exp(logits - m)
    gw = e / jnp.sum(e, axis=-1, keepdims=True)
    maxw = jnp.max(gw, axis=-1, keepdims=True)
    lanes = jax.lax.broadcasted_iota(jnp.int32, gw.shape, 1)
    idx = jnp.min(jnp.where(gw >= maxw, lanes, 128), axis=-1, keepdims=True)
    keep = (maxw < 0.6) | (lanes == idx)
    coef = jnp.where(keep, gw, 0.0)
    coef_ref[...] = coef
    b2e_ref[...] = (coef[:, 0:1] * b2s_ref[0:1, :]
                    + coef[:, 1:2] * b2s_ref[1:2, :]
                    + coef[:, 2:3] * b2s_ref[2:3, :])


def _mlp_body(coef_ref, x_ref, fw1_ref, fw2_ref, cw1_ref, cw2_ref,
              sw1_ref, sw2_ref, fb1_ref, cb1_ref, sb1_ref, b2e_ref,
              o_ref, xs_ref):
    s = pl.program_id(0)
    i = pl.program_id(1)
    b = s // ROWS_PER_BATCH

    @pl.when(i == 0)
    def _init():
        xs_ref[...] = x_ref[...].astype(jnp.bfloat16)
        o_ref[...] = jnp.broadcast_to(b2e_ref[pl.ds(b, 1), :], (BS, H))

    def contrib(w1_ref, w2_ref, b1_ref, path):
        sc = coef_ref[b, path]
        h = jnp.dot(xs_ref[...], w1_ref[...].astype(jnp.bfloat16),
                    preferred_element_type=jnp.float32)
        h = h + b1_ref[...]
        h = h * jax.nn.sigmoid(h)
        h = h * sc
        o_ref[...] += jnp.dot(h.astype(jnp.bfloat16),
                              w2_ref[...].astype(jnp.bfloat16),
                              preferred_element_type=jnp.float32)

    @pl.when(i < NF)
    def _full():
        contrib(fw1_ref, fw2_ref, fb1_ref, 0)

    @pl.when((i >= NF) & (i < NF + NC))
    def _comp():
        contrib(cw1_ref, cw2_ref, cb1_ref, 1)

    @pl.when(i >= NF + NC)
    def _sparse():
        contrib(sw1_ref, sw2_ref, sb1_ref, 2)


def kernel(hidden_states, gate_w1, gate_b1, gate_w2, gate_b2,
           full_w1, full_b1, full_w2, full_b2,
           comp_w1, comp_b1, comp_w2, comp_b2,
           sparse_w1, sparse_b1, sparse_w2, sparse_b2):
    x = hidden_states.reshape(B * S, H)
    gw2p = jnp.pad(gate_w2, ((0, 0), (0, 128 - 3)))
    gb2p = jnp.pad(gate_b2, ((0, 128 - 3),), constant_values=-1e9)
    b2s = jnp.stack([full_b2, comp_b2, sparse_b2], axis=0)

    coef, b2e = pl.pallas_call(
        _gate_body,
        grid=(1,),
        in_specs=[
            pl.BlockSpec((B * S, H), lambda i: (0, 0)),
            pl.BlockSpec((H, G), lambda i: (0, 0)),
            pl.BlockSpec((1, G), lambda i: (0, 0)),
            pl.BlockSpec((G, 128), lambda i: (0, 0)),
            pl.BlockSpec((1, 128), lambda i: (0, 0)),
            pl.BlockSpec((3, H), lambda i: (0, 0)),
        ],
        out_specs=[
            pl.BlockSpec((B, 128), lambda i: (0, 0)),
            pl.BlockSpec((B, H), lambda i: (0, 0)),
        ],
        out_shape=[
            jax.ShapeDtypeStruct((B, 128), jnp.float32),
            jax.ShapeDtypeStruct((B, H), jnp.float32),
        ],
    )(x, gate_w1, gate_b1.reshape(1, G), gw2p, gb2p.reshape(1, 128), b2s)

    out = pl.pallas_call(
        _mlp_body,
        grid=(NS, NI),
        in_specs=[
            pl.BlockSpec(memory_space=pltpu.SMEM),
            pl.BlockSpec((BS, H), lambda s, i: (s, 0)),
            pl.BlockSpec((H, BI), lambda s, i: (0, jnp.minimum(i, NF - 1))),
            pl.BlockSpec((BI, H), lambda s, i: (jnp.minimum(i, NF - 1), 0)),
            pl.BlockSpec((H, BI), lambda s, i: (0, jnp.clip(i - NF, 0, NC - 1))),
            pl.BlockSpec((BI, H), lambda s, i: (jnp.clip(i - NF, 0, NC - 1), 0)),
            pl.BlockSpec((H, BI), lambda s, i: (0, jnp.clip(i - NF - NC, 0, NF - 1))),
            pl.BlockSpec((BI, H), lambda s, i: (jnp.clip(i - NF - NC, 0, NF - 1), 0)),
            pl.BlockSpec((1, BI), lambda s, i: (0, jnp.minimum(i, NF - 1))),
            pl.BlockSpec((1, BI), lambda s, i: (0, jnp.clip(i - NF, 0, NC - 1))),
            pl.BlockSpec((1, BI), lambda s, i: (0, jnp.clip(i - NF - NC, 0, NF - 1))),
            pl.BlockSpec((B, H), lambda s, i: (0, 0)),
        ],
        out_specs=pl.BlockSpec((BS, H), lambda s, i: (s, 0)),
        out_shape=jax.ShapeDtypeStruct((B * S, H), jnp.float32),
        scratch_shapes=[pltpu.VMEM((BS, H), jnp.bfloat16)],
        compiler_params=pltpu.CompilerParams(
            dimension_semantics=("parallel", "arbitrary")),
    )(coef, x, full_w1, full_w2, comp_w1, comp_w2, sparse_w1, sparse_w2,
      full_b1.reshape(1, IF), comp_b1.reshape(1, IC),
      sparse_b1.reshape(1, IF), b2e)

    return out.reshape(B, S, H)


# speedup vs baseline: 1.0167x; 1.0167x over previous
import jax
import jax.numpy as jnp
from jax.experimental import pallas as pl
from jax.experimental.pallas import tpu as pltpu

B, S, H, G = 2, 2048, 1024, 256
IF, IC = 4096, 2048
ITOT = IF + IC + IF
BS = 1024
BI = 512
NS = (B * S) // BS
NF = IF // BI
NC = IC // BI
NI = NF + NC + NF
ROWS_PER_BATCH = S // BS


def _gate_body(x_ref, gw1_ref, gb1_ref, gw2_ref, gb2_ref, b2s_ref,
               coef_ref, b2e_ref):
    gf0 = jnp.mean(x_ref[0:S, :], axis=0, keepdims=True)
    gf1 = jnp.mean(x_ref[S:2 * S, :], axis=0, keepdims=True)
    gf = jnp.concatenate([gf0, gf1], axis=0)
    gh = jnp.maximum(
        jnp.dot(gf, gw1_ref[...], preferred_element_type=jnp.float32)
        + gb1_ref[...], 0.0)
    logits = (jnp.dot(gh, gw2_ref[...], preferred_element_type=jnp.float32)
              + gb2_ref[...])
    m = jnp.max(logits, axis=-1, keepdims=True)
    e = jnp.exp(logits - m)
    gw = e / jnp.sum(e, axis=-1, keepdims=True)
    maxw = jnp.max(gw, axis=-1, keepdims=True)
    lanes = jax.lax.broadcasted_iota(jnp.int32, gw.shape, 1)
    idx = jnp.min(jnp.where(gw >= maxw, lanes, 128), axis=-1, keepdims=True)
    keep = (maxw < 0.6) | (lanes == idx)
    coef = jnp.where(keep, gw, 0.0)
    coef_ref[...] = coef
    b2e_ref[...] = (coef[:, 0:1] * b2s_ref[0:1, :]
                    + coef[:, 1:2] * b2s_ref[1:2, :]
                    + coef[:, 2:3] * b2s_ref[2:3, :])


def _mlp_body(coef_ref, x_ref, fw1_ref, fw2_ref, cw1_ref, cw2_ref,
              sw1_ref, sw2_ref, fb1_ref, cb1_ref, sb1_ref, b2e_ref,
              o_ref, xs_ref):
    s = pl.program_id(0)
    i = pl.program_id(1)
    b = s // ROWS_PER_BATCH

    @pl.when(i == 0)
    def _init():
        xs_ref[...] = x_ref[...].astype(jnp.bfloat16)
        o_ref[...] = jnp.broadcast_to(b2e_ref[pl.ds(b, 1), :], (BS, H))

    def contrib(w1_ref, w2_ref, b1_ref, path):
        sc = coef_ref[b, path]
        h = jnp.dot(xs_ref[...], w1_ref[...].astype(jnp.bfloat16),
                    preferred_element_type=jnp.float32)
        h = h + b1_ref[...]
        h = (0.5 * sc) * h * (jnp.tanh(0.5 * h) + 1.0)
        o_ref[...] += jnp.dot(h.astype(jnp.bfloat16),
                              w2_ref[...].astype(jnp.bfloat16),
                              preferred_element_type=jnp.float32)

    @pl.when(i < NF)
    def _full():
        contrib(fw1_ref, fw2_ref, fb1_ref, 0)

    @pl.when((i >= NF) & (i < NF + NC))
    def _comp():
        contrib(cw1_ref, cw2_ref, cb1_ref, 1)

    @pl.when(i >= NF + NC)
    def _sparse():
        contrib(sw1_ref, sw2_ref, sb1_ref, 2)


def kernel(hidden_states, gate_w1, gate_b1, gate_w2, gate_b2,
           full_w1, full_b1, full_w2, full_b2,
           comp_w1, comp_b1, comp_w2, comp_b2,
           sparse_w1, sparse_b1, sparse_w2, sparse_b2):
    x = hidden_states.reshape(B * S, H)
    gw2p = jnp.pad(gate_w2, ((0, 0), (0, 128 - 3)))
    gb2p = jnp.pad(gate_b2, ((0, 128 - 3),), constant_values=-1e9)
    b2s = jnp.stack([full_b2, comp_b2, sparse_b2], axis=0)

    coef, b2e = pl.pallas_call(
        _gate_body,
        grid=(1,),
        in_specs=[
            pl.BlockSpec((B * S, H), lambda i: (0, 0)),
            pl.BlockSpec((H, G), lambda i: (0, 0)),
            pl.BlockSpec((1, G), lambda i: (0, 0)),
            pl.BlockSpec((G, 128), lambda i: (0, 0)),
            pl.BlockSpec((1, 128), lambda i: (0, 0)),
            pl.BlockSpec((3, H), lambda i: (0, 0)),
        ],
        out_specs=[
            pl.BlockSpec((B, 128), lambda i: (0, 0)),
            pl.BlockSpec((B, H), lambda i: (0, 0)),
        ],
        out_shape=[
            jax.ShapeDtypeStruct((B, 128), jnp.float32),
            jax.ShapeDtypeStruct((B, H), jnp.float32),
        ],
    )(x, gate_w1, gate_b1.reshape(1, G), gw2p, gb2p.reshape(1, 128), b2s)

    out = pl.pallas_call(
        _mlp_body,
        grid=(NS, NI),
        in_specs=[
            pl.BlockSpec(memory_space=pltpu.SMEM),
            pl.BlockSpec((BS, H), lambda s, i: (s, 0)),
            pl.BlockSpec((H, BI), lambda s, i: (0, jnp.minimum(i, NF - 1))),
            pl.BlockSpec((BI, H), lambda s, i: (jnp.minimum(i, NF - 1), 0)),
            pl.BlockSpec((H, BI), lambda s, i: (0, jnp.clip(i - NF, 0, NC - 1))),
            pl.BlockSpec((BI, H), lambda s, i: (jnp.clip(i - NF, 0, NC - 1), 0)),
            pl.BlockSpec((H, BI), lambda s, i: (0, jnp.clip(i - NF - NC, 0, NF - 1))),
            pl.BlockSpec((BI, H), lambda s, i: (jnp.clip(i - NF - NC, 0, NF - 1), 0)),
            pl.BlockSpec((1, BI), lambda s, i: (0, jnp.minimum(i, NF - 1))),
            pl.BlockSpec((1, BI), lambda s, i: (0, jnp.clip(i - NF, 0, NC - 1))),
            pl.BlockSpec((1, BI), lambda s, i: (0, jnp.clip(i - NF - NC, 0, NF - 1))),
            pl.BlockSpec((B, H), lambda s, i: (0, 0)),
        ],
        out_specs=pl.BlockSpec((BS, H), lambda s, i: (s, 0)),
        out_shape=jax.ShapeDtypeStruct((B * S, H), jnp.float32),
        scratch_shapes=[pltpu.VMEM((BS, H), jnp.bfloat16)],
        compiler_params=pltpu.CompilerParams(
            dimension_semantics=("parallel", "arbitrary")),
    )(coef, x, full_w1, full_w2, comp_w1, comp_w2, sparse_w1, sparse_w2,
      full_b1.reshape(1, IF), comp_b1.reshape(1, IC),
      sparse_b1.reshape(1, IF), b2e)

    return out.reshape(B, S, H)
